# baseline (device time: 30711 ns/iter reference)
import jax
import jax.numpy as jnp
from jax import lax
from jax.experimental import pallas as pl
from jax.experimental.pallas import tpu as pltpu

N_CHUNKS = 8


def kernel(x, W, labels):
    T, D = x.shape
    _, V = W.shape
    VC = V // N_CHUNKS

    def body(
        x_hbm, w_hbm, lbl_ref, out_ref,
        xv, wv, comm_ref, x_sem, copy_sems, send_sem, recv_sem,
    ):
        my_x = lax.axis_index("x")
        my_y = lax.axis_index("y")
        my_z = lax.axis_index("z")
        partner = (1 - my_x, my_y, my_z)

        barrier_sem = pltpu.get_barrier_semaphore()
        pl.semaphore_signal(
            barrier_sem, inc=1, device_id=partner,
            device_id_type=pl.DeviceIdType.MESH,
        )

        xcp = pltpu.make_async_copy(x_hbm, xv, x_sem)
        xcp.start()
        copies = []
        for c in range(N_CHUNKS):
            cp = pltpu.make_async_copy(
                w_hbm.at[:, pl.ds(c * VC, VC)],
                wv.at[:, pl.ds(c * VC, VC)],
                copy_sems.at[c],
            )
            cp.start()
            copies.append(cp)
        xcp.wait()

        xvv = xv[:, :]
        lbl = lbl_ref[:]
        col = lax.broadcasted_iota(jnp.int32, (T, VC), 1)

        def stats(l, c):
            e = jnp.sum(jnp.exp(l), axis=1)
            lbl_local = lbl - (my_x * V + c * VC)
            gp = jnp.sum(
                jnp.where(col == lbl_local[:, None], l, 0.0), axis=1
            )
            return e, gp

        s_loc = None
        g_loc = None
        l_prev = None
        for c in range(N_CHUNKS):
            copies[c].wait()
            l_c = jnp.dot(
                xvv, wv[:, c * VC:(c + 1) * VC],
                preferred_element_type=jnp.float32,
            )
            if l_prev is not None:
                e, gp = stats(l_prev, c - 1)
                s_loc = e if s_loc is None else s_loc + e
                g_loc = gp if g_loc is None else g_loc + gp
            l_prev = l_c
        e, gp = stats(l_prev, N_CHUNKS - 1)
        s_loc = s_loc + e
        g_loc = g_loc + gp

        pl.semaphore_wait(barrier_sem, 1)
        comm_ref[0, 0, :] = s_loc
        comm_ref[0, 1, :] = g_loc
        rdma = pltpu.make_async_remote_copy(
            src_ref=comm_ref.at[0],
            dst_ref=comm_ref.at[1],
            send_sem=send_sem,
            recv_sem=recv_sem,
            device_id=partner,
            device_id_type=pl.DeviceIdType.MESH,
        )
        rdma.start()
        rdma.wait()

        s_tot = s_loc + comm_ref[1, 0, :]
        g_tot = g_loc + comm_ref[1, 1, :]
        out_ref[:] = jnp.log(s_tot) - g_tot

    return pl.pallas_call(
        body,
        out_shape=jax.ShapeDtypeStruct((T,), jnp.float32),
        in_specs=[
            pl.BlockSpec(memory_space=pl.ANY),
            pl.BlockSpec(memory_space=pl.ANY),
            pl.BlockSpec(memory_space=pltpu.VMEM),
        ],
        out_specs=pl.BlockSpec(memory_space=pltpu.VMEM),
        scratch_shapes=[
            pltpu.VMEM((T, D), jnp.float32),
            pltpu.VMEM((D, V), jnp.float32),
            pltpu.VMEM((2, 2, T), jnp.float32),
            pltpu.SemaphoreType.DMA,
            pltpu.SemaphoreType.DMA((N_CHUNKS,)),
            pltpu.SemaphoreType.DMA,
            pltpu.SemaphoreType.DMA,
        ],
        compiler_params=pltpu.CompilerParams(
            collective_id=0,
            vmem_limit_bytes=100 * 1024 * 1024,
        ),
    )(x, W, labels)


# device time: 23641 ns/iter; 1.2991x vs baseline; 1.2991x over previous
import jax
import jax.numpy as jnp
from jax import lax
from jax.experimental import pallas as pl
from jax.experimental.pallas import tpu as pltpu

N_CHUNKS = 8


def kernel(x, W, labels):
    T, D = x.shape
    _, V = W.shape
    VC = V // N_CHUNKS

    def body(
        x_ref, w_hbm, lbl_ref, out_ref,
        wv, comm_ref, copy_sems, send_sem, recv_sem,
    ):
        my_x = lax.axis_index("x")
        my_y = lax.axis_index("y")
        my_z = lax.axis_index("z")
        partner = (1 - my_x, my_y, my_z)

        barrier_sem = pltpu.get_barrier_semaphore()
        pl.semaphore_signal(
            barrier_sem, inc=1, device_id=partner,
            device_id_type=pl.DeviceIdType.MESH,
        )

        copies = []
        for c in range(N_CHUNKS):
            cp = pltpu.make_async_copy(
                w_hbm.at[:, pl.ds(c * VC, VC)],
                wv.at[:, pl.ds(c * VC, VC)],
                copy_sems.at[c],
            )
            cp.start()
            copies.append(cp)

        xv = x_ref[:, :]
        lbl = lbl_ref[:]
        col = lax.broadcasted_iota(jnp.int32, (T, VC), 1)

        s_loc = None
        g_loc = None
        for c in range(N_CHUNKS):
            copies[c].wait()
            logits_c = jnp.dot(
                xv, wv[:, c * VC:(c + 1) * VC],
                preferred_element_type=jnp.float32,
            )
            e = jnp.sum(jnp.exp(logits_c), axis=1)
            lbl_local = lbl - (my_x * V + c * VC)
            gp = jnp.sum(
                jnp.where(col == lbl_local[:, None], logits_c, 0.0), axis=1
            )
            s_loc = e if s_loc is None else s_loc + e
            g_loc = gp if g_loc is None else g_loc + gp
            if c == 0:
                pl.semaphore_wait(barrier_sem, 1)

        comm_ref[0, 0, :] = s_loc
        comm_ref[0, 1, :] = g_loc
        rdma = pltpu.make_async_remote_copy(
            src_ref=comm_ref.at[0],
            dst_ref=comm_ref.at[1],
            send_sem=send_sem,
            recv_sem=recv_sem,
            device_id=partner,
            device_id_type=pl.DeviceIdType.MESH,
        )
        rdma.start()
        rdma.wait_recv()

        s_tot = s_loc + comm_ref[1, 0, :]
        g_tot = g_loc + comm_ref[1, 1, :]
        out_ref[:] = jnp.log(s_tot) - g_tot
        rdma.wait_send()

    return pl.pallas_call(
        body,
        out_shape=jax.ShapeDtypeStruct((T,), jnp.float32),
        in_specs=[
            pl.BlockSpec(memory_space=pltpu.VMEM),
            pl.BlockSpec(memory_space=pl.ANY),
            pl.BlockSpec(memory_space=pltpu.VMEM),
        ],
        out_specs=pl.BlockSpec(memory_space=pltpu.VMEM),
        scratch_shapes=[
            pltpu.VMEM((D, V), jnp.float32),
            pltpu.VMEM((2, 2, T), jnp.float32),
            pltpu.SemaphoreType.DMA((N_CHUNKS,)),
            pltpu.SemaphoreType.DMA,
            pltpu.SemaphoreType.DMA,
        ],
        compiler_params=pltpu.CompilerParams(
            collective_id=0,
            vmem_limit_bytes=100 * 1024 * 1024,
        ),
    )(x, W, labels)


# device time: 23210 ns/iter; 1.3232x vs baseline; 1.0186x over previous
import jax
import jax.numpy as jnp
from jax import lax
from jax.experimental import pallas as pl
from jax.experimental.pallas import tpu as pltpu

N_CHUNKS = 8


def kernel(x, W, labels):
    T, D = x.shape
    _, V = W.shape
    VC = V // N_CHUNKS

    def body(
        x_ref, w_hbm, lbl_ref, out_ref,
        wv, comm_ref, copy_sems, send_sem, recv_sem,
    ):
        my_x = lax.axis_index("x")
        my_y = lax.axis_index("y")
        my_z = lax.axis_index("z")
        partner = (1 - my_x, my_y, my_z)

        barrier_sem = pltpu.get_barrier_semaphore()
        pl.semaphore_signal(
            barrier_sem, inc=1, device_id=partner,
            device_id_type=pl.DeviceIdType.MESH,
        )

        copies = []
        for c in range(N_CHUNKS):
            cp = pltpu.make_async_copy(
                w_hbm.at[:, pl.ds(c * VC, VC)],
                wv.at[:, pl.ds(c * VC, VC)],
                copy_sems.at[c],
            )
            cp.start()
            copies.append(cp)

        xv = x_ref[:, :]
        lbl = lbl_ref[:]
        col = lax.broadcasted_iota(jnp.int32, (T, VC), 1)

        s128 = None
        g128 = None
        for c in range(N_CHUNKS):
            copies[c].wait()
            logits_c = jnp.dot(
                xv, wv[:, c * VC:(c + 1) * VC],
                preferred_element_type=jnp.float32,
            )
            E = jnp.exp(logits_c)
            lbl_local = lbl - (my_x * V + c * VC)
            Mk = jnp.where(col == lbl_local[:, None], logits_c, 0.0)
            e128 = E[:, 0:128]
            g128c = Mk[:, 0:128]
            for j in range(1, VC // 128):
                e128 = e128 + E[:, j * 128:(j + 1) * 128]
                g128c = g128c + Mk[:, j * 128:(j + 1) * 128]
            s128 = e128 if s128 is None else s128 + e128
            g128 = g128c if g128 is None else g128 + g128c
            if c == 0:
                pl.semaphore_wait(barrier_sem, 1)

        s_loc = jnp.sum(s128, axis=1)
        g_loc = jnp.sum(g128, axis=1)
        comm_ref[0, 0, :] = s_loc
        comm_ref[0, 1, :] = g_loc
        rdma = pltpu.make_async_remote_copy(
            src_ref=comm_ref.at[0],
            dst_ref=comm_ref.at[1],
            send_sem=send_sem,
            recv_sem=recv_sem,
            device_id=partner,
            device_id_type=pl.DeviceIdType.MESH,
        )
        rdma.start()
        rdma.wait_recv()

        s_tot = s_loc + comm_ref[1, 0, :]
        g_tot = g_loc + comm_ref[1, 1, :]
        out_ref[:] = jnp.log(s_tot) - g_tot
        rdma.wait_send()

    return pl.pallas_call(
        body,
        out_shape=jax.ShapeDtypeStruct((T,), jnp.float32),
        in_specs=[
            pl.BlockSpec(memory_space=pltpu.VMEM),
            pl.BlockSpec(memory_space=pl.ANY),
            pl.BlockSpec(memory_space=pltpu.VMEM),
        ],
        out_specs=pl.BlockSpec(memory_space=pltpu.VMEM),
        scratch_shapes=[
            pltpu.VMEM((D, V), jnp.float32),
            pltpu.VMEM((2, 2, T), jnp.float32),
            pltpu.SemaphoreType.DMA((N_CHUNKS,)),
            pltpu.SemaphoreType.DMA,
            pltpu.SemaphoreType.DMA,
        ],
        compiler_params=pltpu.CompilerParams(
            collective_id=0,
            vmem_limit_bytes=100 * 1024 * 1024,
        ),
    )(x, W, labels)
